# Initial kernel scaffold; baseline (speedup 1.0000x reference)
import jax
import jax.numpy as jnp
from jax.experimental import pallas as pl


def kernel(positions, values, motion_scores, static_mem, confidence):
    raise NotImplementedError("write your pallas kernel here")


# speedup vs baseline: 6.4326x; 6.4326x over previous
import functools

import jax
import jax.numpy as jnp
from jax import lax
from jax.experimental import pallas as pl
from jax.experimental.pallas import tpu as pltpu
from jax.experimental.pallas import tpu_sc as plsc

H_DIM, W_DIM = 512, 512
C_DIM = 128
N_OBS = 131072
DECAY = 0.95
THRESH = 0.1

NC, NS = 2, 16
NW = NC * NS
SLOTS = H_DIM * W_DIM
SPT = SLOTS // NW
SPT_SHIFT = 13
CHUNK = 4096
NCHUNK = N_OBS // CHUNK
GRP = CHUNK // 16
ROWCH = 64
CAP = SPT + ROWCH

_mesh = plsc.VectorSubcoreMesh(
    core_axis_name="c", subcore_axis_name="s", num_cores=NC, num_subcores=NS)


@functools.partial(
    pl.kernel,
    mesh=_mesh,
    compiler_params=pltpu.CompilerParams(needs_layout_passes=False),
    scratch_types=[
        pltpu.VMEM((SPT,), jnp.int32),
        pltpu.VMEM((CHUNK,), jnp.int32),
        pltpu.VMEM((CHUNK,), jnp.int32),
        pltpu.VMEM((CHUNK,), jnp.float32),
        pltpu.VMEM((CAP,), jnp.int32),
        pltpu.VMEM((CAP,), jnp.int32),
        pltpu.VMEM((SPT,), jnp.float32),
        pltpu.VMEM((ROWCH,), jnp.int32),
        pltpu.VMEM((ROWCH,), jnp.int32),
        pltpu.VMEM((ROWCH, C_DIM), jnp.float32),
        pltpu.VMEM((ROWCH, C_DIM), jnp.float32),
        pltpu.SemaphoreType.DMA,
        pltpu.SemaphoreType.DMA,
    ],
)
def _sc_update(u_hbm, v_hbm, mo_hbm, val_hbm, mem_hbm, conf_hbm,
               winner, u_buf, v_buf, mo_buf, slot_list, j_list, conf_shard,
               slot_buf, j_buf, mem_rows, val_rows, sem0, sem1):
    wid = lax.axis_index("s") * NC + lax.axis_index("c")
    lane = lax.iota(jnp.int32, 16)
    lane2 = lane * 2
    zeros16 = jnp.zeros((16,), jnp.int32)

    def _init(g, _):
        winner[pl.ds(g * 16, 16)] = zeros16
        return 0
    lax.fori_loop(0, SPT // 16, _init, 0, unroll=4)

    def _chunk(ci, _):
        base = ci * CHUNK
        pltpu.sync_copy(u_hbm.at[pl.ds(base, CHUNK)], u_buf)
        pltpu.sync_copy(v_hbm.at[pl.ds(base, CHUNK)], v_buf)
        pltpu.sync_copy(mo_hbm.at[pl.ds(base, CHUNK)], mo_buf)

        def _grp(g, _):
            off = g * 16
            uu = u_buf[pl.ds(off, 16)]
            vv = v_buf[pl.ds(off, 16)]
            idx = uu * W_DIM + vv
            local = lax.bitwise_and(idx, SPT - 1)
            owner = lax.shift_right_logical(idx, SPT_SHIFT)
            match = owner == wid
            mo = mo_buf[pl.ds(off, 16)]
            flag = jnp.where(mo < THRESH, 1, 0)
            enc = (2 * (base + off) + 2) + lane2 + flag
            plsc.store_scatter(winner, [local], enc, mask=match)
            return 0
        lax.fori_loop(0, GRP, _grp, 0)
        return 0
    lax.fori_loop(0, NCHUNK, _chunk, 0)

    shard_base = wid * SPT
    pltpu.sync_copy(conf_hbm.at[pl.ds(shard_base, SPT)], conf_shard)

    def _compact(g, k):
        off = g * 16
        w = winner[pl.ds(off, 16)]
        valid = lax.bitwise_and(w, 1) == 1
        slots_g = (shard_base + off) + lane
        j_g = lax.shift_right_logical(w, 1) - 1
        plsc.store_compressed(slot_list.at[pl.ds(k, 16)], slots_g, mask=valid)
        plsc.store_compressed(j_list.at[pl.ds(k, 16)], j_g, mask=valid)
        c = conf_shard[pl.ds(off, 16)]
        conf_shard[pl.ds(off, 16)] = jnp.where(
            valid, jnp.minimum(c + 0.1, 1.0), c)
        return k + jnp.sum(jnp.where(valid, 1, 0))
    k = lax.fori_loop(0, SPT // 16, _compact, jnp.int32(0))

    pltpu.sync_copy(conf_shard, conf_hbm.at[pl.ds(shard_base, SPT)])

    @pl.when(k > 0)
    def _pad():
        lastv = jnp.full((16,), k - 1, jnp.int32)
        bslot = plsc.load_gather(slot_list, [lastv])
        bj = plsc.load_gather(j_list, [lastv])
        for t in range(ROWCH // 16):
            slot_list[pl.ds(k + t * 16, 16)] = bslot
            j_list[pl.ds(k + t * 16, 16)] = bj

    nch = lax.div(k + (ROWCH - 1), jnp.int32(ROWCH))

    def _rows(t, _):
        off = t * ROWCH
        for q in range(ROWCH // 16):
            slot_buf[pl.ds(q * 16, 16)] = slot_list[pl.ds(off + q * 16, 16)]
            j_buf[pl.ds(q * 16, 16)] = j_list[pl.ds(off + q * 16, 16)]
        cp0 = pltpu.make_async_copy(mem_hbm.at[slot_buf], mem_rows, sem0)
        cp1 = pltpu.make_async_copy(val_hbm.at[j_buf], val_rows, sem1)
        cp0.start()
        cp1.start()
        cp0.wait()
        cp1.wait()

        def _ema(i, _):
            r = lax.shift_right_logical(i, 3)
            cc = lax.bitwise_and(i, 7) * 16
            m = mem_rows[r, pl.ds(cc, 16)]
            vl = val_rows[r, pl.ds(cc, 16)]
            mem_rows[r, pl.ds(cc, 16)] = DECAY * m + (1.0 - DECAY) * vl
            return 0
        lax.fori_loop(0, ROWCH * C_DIM // 16, _ema, 0, unroll=8)

        wb = pltpu.make_async_copy(mem_rows, mem_hbm.at[slot_buf], sem0)
        wb.start()
        wb.wait()
        return 0
    lax.fori_loop(0, nch, _rows, 0)


def kernel(positions, values, motion_scores, static_mem, confidence):
    u = positions[:, 0].astype(jnp.int32)
    v = positions[:, 1].astype(jnp.int32)
    mem_ref = jax.new_ref(static_mem)
    conf_ref = jax.new_ref(confidence)
    _sc_update(u, v, motion_scores, values, mem_ref, conf_ref)
    return mem_ref[...], conf_ref[...]


# speedup vs baseline: 8.5906x; 1.3355x over previous
import functools

import jax
import jax.numpy as jnp
from jax import lax
from jax.experimental import pallas as pl
from jax.experimental.pallas import tpu as pltpu
from jax.experimental.pallas import tpu_sc as plsc

H_DIM, W_DIM = 512, 512
C_DIM = 128
N_OBS = 131072
DECAY = 0.95
THRESH = 0.1

NC, NS = 2, 16
NW = NC * NS
SLOTS = H_DIM * W_DIM
SPT = SLOTS // NW
SPT_SHIFT = 13
CHUNK = 4096
NCHUNK = N_OBS // CHUNK
GRP = CHUNK // 16
ROWCH = 64
CAP = SPT + ROWCH

_mesh = plsc.VectorSubcoreMesh(
    core_axis_name="c", subcore_axis_name="s", num_cores=NC, num_subcores=NS)


@functools.partial(
    pl.kernel,
    mesh=_mesh,
    compiler_params=pltpu.CompilerParams(needs_layout_passes=False),
    scratch_types=[
        pltpu.VMEM((SPT,), jnp.int32),
        pltpu.VMEM((CHUNK,), jnp.int32),
        pltpu.VMEM((CHUNK,), jnp.int32),
        pltpu.VMEM((CHUNK,), jnp.float32),
        pltpu.VMEM((CHUNK,), jnp.int32),
        pltpu.VMEM((CHUNK,), jnp.int32),
        pltpu.VMEM((CHUNK,), jnp.float32),
        pltpu.SemaphoreType.DMA,
        pltpu.SemaphoreType.DMA,
        pltpu.VMEM((CAP,), jnp.int32),
        pltpu.VMEM((CAP,), jnp.int32),
        pltpu.VMEM((SPT,), jnp.float32),
        pltpu.VMEM((ROWCH,), jnp.int32),
        pltpu.VMEM((ROWCH,), jnp.int32),
        pltpu.VMEM((ROWCH, C_DIM), jnp.float32),
        pltpu.VMEM((ROWCH, C_DIM), jnp.float32),
        pltpu.SemaphoreType.DMA,
        pltpu.SemaphoreType.DMA,
    ],
)
def _sc_update(u_hbm, v_hbm, mo_hbm, val_hbm, mem_hbm, conf_hbm,
               winner, u_buf0, v_buf0, mo_buf0, u_buf1, v_buf1, mo_buf1,
               psem0, psem1, slot_list, j_list, conf_shard,
               slot_buf, j_buf, mem_rows, val_rows, sem0, sem1):
    wid = lax.axis_index("s") * NC + lax.axis_index("c")
    lane = lax.iota(jnp.int32, 16)
    lane2 = lane * 2
    zeros16 = jnp.zeros((16,), jnp.int32)

    def _init(g, _):
        winner[pl.ds(g * 16, 16)] = zeros16
        return 0
    lax.fori_loop(0, SPT // 16, _init, 0, unroll=4)

    bufs = ((u_buf0, v_buf0, mo_buf0, psem0), (u_buf1, v_buf1, mo_buf1, psem1))

    def _copies(ci, b):
        base = ci * CHUNK
        ub, vb, mb, ps = bufs[b]
        return (pltpu.make_async_copy(u_hbm.at[pl.ds(base, CHUNK)], ub, ps),
                pltpu.make_async_copy(v_hbm.at[pl.ds(base, CHUNK)], vb, ps),
                pltpu.make_async_copy(mo_hbm.at[pl.ds(base, CHUNK)], mb, ps))

    def _start(ci, b):
        for cp in _copies(ci, b):
            cp.start()

    def _wait(ci, b):
        for cp in _copies(ci, b):
            cp.wait()

    def _scan(ci, b):
        ub, vb, mb, _ = bufs[b]
        base = ci * CHUNK

        def _grp(g, _):
            off = g * 16
            uu = ub[pl.ds(off, 16)]
            vv = vb[pl.ds(off, 16)]
            idx = uu * W_DIM + vv
            local = lax.bitwise_and(idx, SPT - 1)
            owner = lax.shift_right_logical(idx, SPT_SHIFT)
            match = owner == wid
            mo = mb[pl.ds(off, 16)]
            flag = jnp.where(mo < THRESH, 1, 0)
            enc = (2 * (base + off) + 2) + lane2 + flag
            plsc.store_scatter(winner, [local], enc, mask=match)
            return 0
        lax.fori_loop(0, GRP, _grp, 0, unroll=4)

    _start(0, 0)

    def _pair(t, _):
        ci = t * 2
        _wait(ci, 0)
        _start(ci + 1, 1)
        _scan(ci, 0)
        _wait(ci + 1, 1)

        @pl.when(ci + 2 < NCHUNK)
        def _():
            _start(ci + 2, 0)
        _scan(ci + 1, 1)
        return 0
    lax.fori_loop(0, NCHUNK // 2, _pair, 0)

    shard_base = wid * SPT
    pltpu.sync_copy(conf_hbm.at[pl.ds(shard_base, SPT)], conf_shard)

    def _compact(g, k):
        off = g * 16
        w = winner[pl.ds(off, 16)]
        valid = lax.bitwise_and(w, 1) == 1
        slots_g = (shard_base + off) + lane
        j_g = lax.shift_right_logical(w, 1) - 1
        plsc.store_compressed(slot_list.at[pl.ds(k, 16)], slots_g, mask=valid)
        plsc.store_compressed(j_list.at[pl.ds(k, 16)], j_g, mask=valid)
        c = conf_shard[pl.ds(off, 16)]
        conf_shard[pl.ds(off, 16)] = jnp.where(
            valid, jnp.minimum(c + 0.1, 1.0), c)
        return k + jnp.sum(jnp.where(valid, 1, 0))
    k = lax.fori_loop(0, SPT // 16, _compact, jnp.int32(0))

    pltpu.sync_copy(conf_shard, conf_hbm.at[pl.ds(shard_base, SPT)])

    @pl.when(k > 0)
    def _pad():
        lastv = jnp.full((16,), k - 1, jnp.int32)
        bslot = plsc.load_gather(slot_list, [lastv])
        bj = plsc.load_gather(j_list, [lastv])
        for t in range(ROWCH // 16):
            slot_list[pl.ds(k + t * 16, 16)] = bslot
            j_list[pl.ds(k + t * 16, 16)] = bj

    nch = lax.div(k + (ROWCH - 1), jnp.int32(ROWCH))

    def _rows(t, _):
        off = t * ROWCH
        for q in range(ROWCH // 16):
            slot_buf[pl.ds(q * 16, 16)] = slot_list[pl.ds(off + q * 16, 16)]
            j_buf[pl.ds(q * 16, 16)] = j_list[pl.ds(off + q * 16, 16)]
        cp0 = pltpu.make_async_copy(mem_hbm.at[slot_buf], mem_rows, sem0)
        cp1 = pltpu.make_async_copy(val_hbm.at[j_buf], val_rows, sem1)
        cp0.start()
        cp1.start()
        cp0.wait()
        cp1.wait()

        def _ema(i, _):
            r = lax.shift_right_logical(i, 3)
            cc = lax.bitwise_and(i, 7) * 16
            m = mem_rows[r, pl.ds(cc, 16)]
            vl = val_rows[r, pl.ds(cc, 16)]
            mem_rows[r, pl.ds(cc, 16)] = DECAY * m + (1.0 - DECAY) * vl
            return 0
        lax.fori_loop(0, ROWCH * C_DIM // 16, _ema, 0, unroll=8)

        wb = pltpu.make_async_copy(mem_rows, mem_hbm.at[slot_buf], sem0)
        wb.start()
        wb.wait()
        return 0
    lax.fori_loop(0, nch, _rows, 0)


def kernel(positions, values, motion_scores, static_mem, confidence):
    u = positions[:, 0].astype(jnp.int32)
    v = positions[:, 1].astype(jnp.int32)
    mem_ref = jax.new_ref(static_mem)
    conf_ref = jax.new_ref(confidence)
    _sc_update(u, v, motion_scores, values, mem_ref, conf_ref)
    return mem_ref[...], conf_ref[...]
